# baseline (speedup 1.0000x reference)
import functools

import jax
import jax.numpy as jnp
from jax import lax
from jax.experimental import pallas as pl
from jax.experimental.pallas import tpu as pltpu
from jax.experimental.pallas import tpu_sc as plsc

VOCAB = 100000
ENC = 200
EMB = 316
EMB_P = 384
OUT = 1024
BATCH = 4096

NC = 2
NS = 16
NW = NC * NS
NIDX = BATCH * ENC
PER_W = NIDX // NW
CH = 128
NCH = PER_W // CH


def _gather_kernel(idx_hbm, tab_hbm, out_hbm, idx_v, rows0, rows1, g0, g1):
    wid = lax.axis_index("s") * NC + lax.axis_index("c")
    base = wid * PER_W
    rows = (rows0, rows1)
    gsem = (g0, g1)
    pltpu.sync_copy(idx_hbm.at[wid], idx_v)
    pltpu.async_copy(tab_hbm.at[idx_v.at[0]], rows0, g0)

    def pair(i, carry):
        for b in range(2):
            j = 2 * i + b

            @pl.when(j + 1 < NCH)
            def _fire():
                pltpu.async_copy(
                    tab_hbm.at[idx_v.at[j + 1]], rows[1 - b], gsem[1 - b]
                )

            pltpu.make_async_copy(
                tab_hbm.at[idx_v.at[0]], rows[b], gsem[b]
            ).wait()
            pltpu.sync_copy(rows[b], out_hbm.at[pl.ds(base + j * CH, CH)])
        return carry

    lax.fori_loop(0, NCH // 2, pair, 0)


@functools.partial(
    pl.kernel,
    mesh=plsc.VectorSubcoreMesh(core_axis_name="c", subcore_axis_name="s"),
    out_type=jax.ShapeDtypeStruct((NIDX, EMB_P), jnp.float32),
    scratch_types=[
        pltpu.VMEM((NCH, CH), jnp.int32),
        pltpu.VMEM((CH, EMB_P), jnp.float32),
        pltpu.VMEM((CH, EMB_P), jnp.float32),
        pltpu.SemaphoreType.DMA,
        pltpu.SemaphoreType.DMA,
    ],
)
def _sc_gather(idx_hbm, tab_hbm, out_hbm, idx_v, rows0, rows1, g0, g1):
    _gather_kernel(idx_hbm, tab_hbm, out_hbm, idx_v, rows0, rows1, g0, g1)


BK = 768
BM = 2048
KP = ENC * EMB_P


def _mm_kernel(a_ref, w_ref, bias_ref, o_ref):
    k = pl.program_id(1)
    a = a_ref[...].astype(jnp.bfloat16)
    w = w_ref[...]
    acc = lax.dot_general(
        a, w, (((1,), (1,)), ((), ())), preferred_element_type=jnp.float32
    )

    @pl.when(k == 0)
    def _init():
        o_ref[...] = acc + bias_ref[...]

    @pl.when(k > 0)
    def _acc():
        o_ref[...] += acc


def _tc_matmul(flat, w_pad, bias):
    return pl.pallas_call(
        _mm_kernel,
        grid=(BATCH // BM, KP // BK),
        in_specs=[
            pl.BlockSpec((BM, BK), lambda m, k: (m, k)),
            pl.BlockSpec((OUT, BK), lambda m, k: (0, k)),
            pl.BlockSpec((1, OUT), lambda m, k: (0, 0)),
        ],
        out_specs=pl.BlockSpec((BM, OUT), lambda m, k: (m, 0)),
        out_shape=jax.ShapeDtypeStruct((BATCH, OUT), jnp.float32),
        compiler_params=pltpu.CompilerParams(
            dimension_semantics=("parallel", "arbitrary"),
        ),
    )(flat, w_pad, bias)


def kernel(x, emb_table, W, b):
    idx = x.astype(jnp.int32).reshape(NW, NCH, CH)
    tab_pad = jnp.pad(emb_table, ((0, 0), (0, EMB_P - EMB)))
    w_pad = jnp.pad(
        W.reshape(OUT, ENC, EMB), ((0, 0), (0, 0), (0, EMB_P - EMB))
    ).reshape(OUT, KP).astype(jnp.bfloat16)
    flat = _sc_gather(idx, tab_pad).reshape(BATCH, KP)
    return _tc_matmul(flat, w_pad, b.reshape(1, OUT))


# speedup vs baseline: 1.0616x; 1.0616x over previous
import functools

import jax
import jax.numpy as jnp
from jax import lax
from jax.experimental import pallas as pl
from jax.experimental.pallas import tpu as pltpu
from jax.experimental.pallas import tpu_sc as plsc

VOCAB = 100000
ENC = 200
EMB = 316
EMB_P = 384
OUT = 1024
BATCH = 4096

NC = 2
NS = 16
NW = NC * NS
NIDX = BATCH * ENC
PER_W = NIDX // NW
CH = 128
NCH = PER_W // CH


def _gather_kernel(idx_hbm, tab_hbm, out_hbm, idx_v, rows0, rows1, g0, g1):
    wid = lax.axis_index("s") * NC + lax.axis_index("c")
    base = wid * PER_W
    rows = (rows0, rows1)
    gsem = (g0, g1)
    pltpu.sync_copy(idx_hbm.at[wid], idx_v)
    pltpu.async_copy(tab_hbm.at[idx_v.at[0]], rows0, g0)

    def pair(i, carry):
        for b in range(2):
            j = 2 * i + b

            @pl.when(j + 1 < NCH)
            def _fire():
                pltpu.async_copy(
                    tab_hbm.at[idx_v.at[j + 1]], rows[1 - b], gsem[1 - b]
                )

            pltpu.make_async_copy(
                tab_hbm.at[idx_v.at[0]], rows[b], gsem[b]
            ).wait()
            pltpu.sync_copy(rows[b], out_hbm.at[pl.ds(base + j * CH, CH)])
        return carry

    lax.fori_loop(0, NCH // 2, pair, 0)


@functools.partial(
    pl.kernel,
    mesh=plsc.VectorSubcoreMesh(core_axis_name="c", subcore_axis_name="s"),
    out_type=jax.ShapeDtypeStruct((NIDX, EMB_P), jnp.float32),
    scratch_types=[
        pltpu.VMEM((NCH, CH), jnp.int32),
        pltpu.VMEM((CH, EMB_P), jnp.float32),
        pltpu.VMEM((CH, EMB_P), jnp.float32),
        pltpu.SemaphoreType.DMA,
        pltpu.SemaphoreType.DMA,
    ],
)
def _sc_gather(idx_hbm, tab_hbm, out_hbm, idx_v, rows0, rows1, g0, g1):
    _gather_kernel(idx_hbm, tab_hbm, out_hbm, idx_v, rows0, rows1, g0, g1)


TR = 800


def _tab_prep_kernel(t_ref, o_ref):
    o_ref[:, :EMB] = t_ref[...]
    o_ref[:, EMB:] = jnp.zeros((TR, EMB_P - EMB), jnp.float32)


def _tab_prep(tab):
    return pl.pallas_call(
        _tab_prep_kernel,
        grid=(VOCAB // TR,),
        in_specs=[pl.BlockSpec((TR, EMB), lambda i: (i, 0))],
        out_specs=pl.BlockSpec((TR, EMB_P), lambda i: (i, 0)),
        out_shape=jax.ShapeDtypeStruct((VOCAB, EMB_P), jnp.float32),
    )(tab)


WR = 512
WE = 8


def _w_prep_kernel(w_ref, o_ref):
    o_ref[:, :, :EMB] = w_ref[...].astype(jnp.bfloat16)
    o_ref[:, :, EMB:] = jnp.zeros((WR, WE, EMB_P - EMB), jnp.bfloat16)


def _w_prep(w3):
    return pl.pallas_call(
        _w_prep_kernel,
        grid=(OUT // WR, ENC // WE),
        in_specs=[pl.BlockSpec((WR, WE, EMB), lambda i, j: (i, j, 0))],
        out_specs=pl.BlockSpec((WR, WE, EMB_P), lambda i, j: (i, j, 0)),
        out_shape=jax.ShapeDtypeStruct((OUT, ENC, EMB_P), jnp.bfloat16),
    )(w3)


BK = 1024
BM = 2048
KP = ENC * EMB_P


def _mm_kernel(a_ref, w_ref, bias_ref, o_ref):
    k = pl.program_id(1)
    acc = lax.dot_general(
        a_ref[...].astype(jnp.bfloat16), w_ref[...], (((1,), (1,)), ((), ())),
        preferred_element_type=jnp.float32,
    )

    @pl.when(k == 0)
    def _init():
        o_ref[...] = acc + bias_ref[...]

    @pl.when(k > 0)
    def _acc():
        o_ref[...] += acc


def _tc_matmul(flat, w_pad, bias):
    return pl.pallas_call(
        _mm_kernel,
        grid=(BATCH // BM, KP // BK),
        in_specs=[
            pl.BlockSpec((BM, BK), lambda m, k: (m, k)),
            pl.BlockSpec((OUT, BK), lambda m, k: (0, k)),
            pl.BlockSpec((1, OUT), lambda m, k: (0, 0)),
        ],
        out_specs=pl.BlockSpec((BM, OUT), lambda m, k: (m, 0)),
        out_shape=jax.ShapeDtypeStruct((BATCH, OUT), jnp.float32),
        compiler_params=pltpu.CompilerParams(
            dimension_semantics=("parallel", "arbitrary"),
        ),
    )(flat, w_pad, bias)


def kernel(x, emb_table, W, b):
    idx = x.astype(jnp.int32).reshape(NW, NCH, CH)
    tab_pad = _tab_prep(emb_table)
    w_pad = _w_prep(W.reshape(OUT, ENC, EMB)).reshape(OUT, KP)
    flat = _sc_gather(idx, tab_pad).reshape(BATCH, KP)
    return _tc_matmul(flat, w_pad, b.reshape(1, OUT))
